# baseline (speedup 1.0000x reference)
import jax
import jax.numpy as jnp
from jax.experimental import pallas as pl
from jax.experimental.pallas import tpu as pltpu
from jax.experimental.pallas import tpu_sc as plsc

NUM_E = 8192
DIM = 256
BATCH = 4096
CCOST = 0.25
BB = 256
NB = BATCH // BB
_PREC = jax.lax.Precision.DEFAULT

_SC = plsc.get_sparse_core_info()
_NW = _SC.num_cores * _SC.num_subcores
_BPW = BATCH // _NW


def _vq_body(x_ref, w_ref, wsq_ref, eio_ref, enc_ref, idx_ref, cnt_ref):
    i = pl.program_id(0)

    @pl.when(i == 0)
    def _init():
        cnt_ref[...] = jnp.zeros_like(cnt_ref)

    x = x_ref[...]
    w = w_ref[...]
    xsq = jnp.sum(x * x, axis=1, keepdims=True)
    m = jax.lax.dot_general(x, w, (((1,), (1,)), ((), ())),
                            precision=_PREC,
                            preferred_element_type=jnp.float32)
    d = (xsq + wsq_ref[...]) - 2.0 * m
    dmin = jnp.min(d, axis=1, keepdims=True)
    eio = eio_ref[...]
    idx = jnp.min(jnp.where(d == dmin, eio, NUM_E), axis=1, keepdims=True)
    enc = (eio == idx).astype(jnp.float32)
    enc_ref[...] = enc
    idx_ref[...] = idx
    cnt_ref[...] += jnp.sum(enc.reshape(BB // 8, 8, NUM_E), axis=0)


def _gather_body(w_hbm, idx_hbm, q_hbm, idx_v, rows_v, sem):
    wid = jax.lax.axis_index("s") * _SC.num_cores + jax.lax.axis_index("c")
    base = wid * _BPW
    pltpu.sync_copy(idx_hbm.at[pl.ds(base, _BPW)], idx_v)
    pltpu.async_copy(w_hbm.at[idx_v], rows_v, sem).wait()
    pltpu.sync_copy(rows_v, q_hbm.at[pl.ds(base, _BPW)])


def _epi_body(x_ref, q_ref, cnt_ref, qst_ref, loss_ref, perp_ref):
    x = x_ref[...]
    q = q_ref[...].astype(jnp.bfloat16).astype(jnp.float32)
    qst_ref[...] = x + (q - x)
    diff = q - x
    s = jnp.sum(diff * diff, axis=(0, 1), keepdims=True)
    mean_sq = s / float(BATCH * DIM)
    loss_ref[...] = mean_sq + CCOST * mean_sq
    p = jnp.sum(cnt_ref[...], axis=0, keepdims=True) / float(BATCH)
    ent = jnp.sum(p * jnp.log(p + 1e-10), axis=1, keepdims=True)
    perp_ref[...] = jnp.exp(-ent)


def kernel(inputs, W):
    x = inputs.reshape(BATCH, DIM)
    wsq = jnp.sum(W * W, axis=1).reshape(1, NUM_E)
    eio = jax.lax.broadcasted_iota(jnp.int32, (1, NUM_E), 1)
    enc, idx, cnt = pl.pallas_call(
        _vq_body,
        grid=(NB,),
        in_specs=[
            pl.BlockSpec((BB, DIM), lambda i: (i, 0)),
            pl.BlockSpec((NUM_E, DIM), lambda i: (0, 0)),
            pl.BlockSpec((1, NUM_E), lambda i: (0, 0)),
            pl.BlockSpec((1, NUM_E), lambda i: (0, 0)),
        ],
        out_specs=[
            pl.BlockSpec((BB, NUM_E), lambda i: (i, 0)),
            pl.BlockSpec((BB, 1), lambda i: (i, 0)),
            pl.BlockSpec((8, NUM_E), lambda i: (0, 0)),
        ],
        out_shape=[
            jax.ShapeDtypeStruct((BATCH, NUM_E), jnp.float32),
            jax.ShapeDtypeStruct((BATCH, 1), jnp.int32),
            jax.ShapeDtypeStruct((8, NUM_E), jnp.float32),
        ],
    )(x, W, wsq, eio)

    mesh = plsc.VectorSubcoreMesh(core_axis_name="c", subcore_axis_name="s")
    q = pl.kernel(
        _gather_body,
        out_type=jax.ShapeDtypeStruct((BATCH, DIM), jnp.float32),
        mesh=mesh,
        scratch_types=[
            pltpu.VMEM((_BPW,), jnp.int32),
            pltpu.VMEM((_BPW, DIM), jnp.float32),
            pltpu.SemaphoreType.DMA,
        ],
    )(W, idx.reshape(BATCH))

    qst, loss, perp = pl.pallas_call(
        _epi_body,
        out_shape=[
            jax.ShapeDtypeStruct((BATCH, DIM), jnp.float32),
            jax.ShapeDtypeStruct((1, 1), jnp.float32),
            jax.ShapeDtypeStruct((1, 1), jnp.float32),
        ],
    )(x, q, cnt)
    return (loss[0, 0], qst.reshape(inputs.shape), perp[0, 0], enc)
